# baseline (speedup 1.0000x reference)
import jax
import jax.numpy as jnp
from jax.experimental import pallas as pl
from jax.experimental.pallas import tpu as pltpu


def _vq_body(x_ref, cb_ref, cbt_ref, wdt_ref,
             quant_ref, recon_ref, idx_ref, loss_ref, cn_ref):
    b = pl.program_id(0)
    j = pl.program_id(1)
    first = jnp.logical_and(b == 0, j == 0)

    @pl.when(first)
    def _init():
        cb = cb_ref[...]
        cn_ref[...] = jnp.sum(cb * cb, axis=1, keepdims=True) * 0.25
        loss_ref[...] = jnp.zeros_like(loss_ref)

    xb = x_ref[0]
    zsq = jnp.sum(xb * xb, axis=0, keepdims=True)
    s2 = jnp.dot(cb_ref[...], xb, preferred_element_type=jnp.float32)
    d = (zsq + s2) + cn_ref[...]

    minv = jnp.min(d, axis=0, keepdims=True)
    k = d.shape[0]
    kio = jax.lax.broadcasted_iota(jnp.int32, d.shape, 0)
    idx = jnp.min(jnp.where(d == minv, kio, k), axis=0)
    idx_ref[0, 0, :] = idx

    oh = (kio == idx[None, :]).astype(jnp.bfloat16)
    q = jnp.dot(cbt_ref[...], oh, preferred_element_type=jnp.float32)
    quant_ref[0] = q
    logits = jnp.dot(wdt_ref[...], q, preferred_element_type=jnp.float32)
    recon_ref[0] = jax.nn.sigmoid(logits)
    loss_ref[...] += minv


def kernel(x, codebook, W_dec):
    B, C, H, W = x.shape
    K = codebook.shape[0]
    HW = H * W
    hwb = 1280 if HW % 1280 == 0 else HW
    nj = HW // hwb

    xr = x.reshape(B, C, HW)
    cbm2 = -2.0 * codebook
    cbT = codebook.T.astype(jnp.bfloat16)
    wdT = W_dec.reshape(1, C)

    quant, recon, idx, lacc = pl.pallas_call(
        _vq_body,
        grid=(B, nj),
        in_specs=[
            pl.BlockSpec((1, C, hwb), lambda b, j: (b, 0, j)),
            pl.BlockSpec((K, C), lambda b, j: (0, 0)),
            pl.BlockSpec((C, K), lambda b, j: (0, 0)),
            pl.BlockSpec((1, C), lambda b, j: (0, 0)),
        ],
        out_specs=[
            pl.BlockSpec((1, C, hwb), lambda b, j: (b, 0, j)),
            pl.BlockSpec((1, 1, hwb), lambda b, j: (b, 0, j)),
            pl.BlockSpec((1, 1, hwb), lambda b, j: (b, 0, j)),
            pl.BlockSpec((1, hwb), lambda b, j: (0, 0)),
        ],
        out_shape=[
            jax.ShapeDtypeStruct((B, C, HW), jnp.float32),
            jax.ShapeDtypeStruct((B, 1, HW), jnp.float32),
            jax.ShapeDtypeStruct((B, 1, HW), jnp.int32),
            jax.ShapeDtypeStruct((1, hwb), jnp.float32),
        ],
        scratch_shapes=[pltpu.VMEM((K, 1), jnp.float32)],
    )(xr, cbm2, cbT, wdT)

    emb_loss = (1.25 / (B * HW * C)) * jnp.sum(lacc)
    return (recon.reshape(B, 1, H, W), quant.reshape(B, C, H, W),
            emb_loss, idx.reshape(-1))


# speedup vs baseline: 1.0103x; 1.0103x over previous
import jax
import jax.numpy as jnp
from jax.experimental import pallas as pl
from jax.experimental.pallas import tpu as pltpu


def _vq_body(x_ref, cb_ref, cbt_ref, wdt_ref,
             quant_ref, recon_ref, idx_ref, loss_ref, cn_ref):
    b = pl.program_id(0)
    j = pl.program_id(1)
    first = jnp.logical_and(b == 0, j == 0)

    @pl.when(first)
    def _init():
        cb = cb_ref[...]
        cn_ref[...] = jnp.sum(cb * cb, axis=1, keepdims=True)
        loss_ref[...] = jnp.zeros_like(loss_ref)

    xb = x_ref[0]
    zsq = jnp.sum(xb * xb, axis=0, keepdims=True)
    s = jnp.dot(cb_ref[...], xb, preferred_element_type=jnp.float32)
    d = (zsq - 2.0 * s) + cn_ref[...]

    minv = jnp.min(d, axis=0, keepdims=True)
    k = d.shape[0]
    kio = jax.lax.broadcasted_iota(jnp.int32, d.shape, 0)
    idx = jnp.min(jnp.where(d == minv, kio, k), axis=0)
    idx_ref[0, 0, :] = idx

    oh = (kio == idx[None, :]).astype(jnp.bfloat16)
    q = jnp.dot(cbt_ref[...], oh, preferred_element_type=jnp.float32)
    quant_ref[0] = q
    logits = jnp.dot(wdt_ref[...], q, preferred_element_type=jnp.float32)
    recon_ref[0] = jax.nn.sigmoid(logits)
    loss_ref[...] += minv


def kernel(x, codebook, W_dec):
    B, C, H, W = x.shape
    K = codebook.shape[0]
    HW = H * W
    hwb = 1280 if HW % 1280 == 0 else HW
    nj = HW // hwb

    xr = x.reshape(B, C, HW)
    cbT = codebook.T.astype(jnp.bfloat16)
    wdT = W_dec.reshape(1, C)

    quant, recon, idx, lacc = pl.pallas_call(
        _vq_body,
        grid=(B, nj),
        in_specs=[
            pl.BlockSpec((1, C, hwb), lambda b, j: (b, 0, j)),
            pl.BlockSpec((K, C), lambda b, j: (0, 0)),
            pl.BlockSpec((C, K), lambda b, j: (0, 0)),
            pl.BlockSpec((1, C), lambda b, j: (0, 0)),
        ],
        out_specs=[
            pl.BlockSpec((1, C, hwb), lambda b, j: (b, 0, j)),
            pl.BlockSpec((1, 1, hwb), lambda b, j: (b, 0, j)),
            pl.BlockSpec((1, 1, hwb), lambda b, j: (b, 0, j)),
            pl.BlockSpec((1, hwb), lambda b, j: (0, 0)),
        ],
        out_shape=[
            jax.ShapeDtypeStruct((B, C, HW), jnp.float32),
            jax.ShapeDtypeStruct((B, 1, HW), jnp.float32),
            jax.ShapeDtypeStruct((B, 1, HW), jnp.int32),
            jax.ShapeDtypeStruct((1, hwb), jnp.float32),
        ],
        scratch_shapes=[pltpu.VMEM((K, 1), jnp.float32)],
    )(xr, codebook, cbT, wdT)

    emb_loss = (1.25 / (B * HW * C)) * jnp.sum(lacc)
    return (recon.reshape(B, 1, H, W), quant.reshape(B, C, H, W),
            emb_loss, idx.reshape(-1))
